# baseline (speedup 1.0000x reference)
import functools

import jax
import jax.numpy as jnp
from jax import lax
from jax.experimental import pallas as pl
from jax.experimental.pallas import tpu as pltpu
from jax.experimental.pallas import tpu_sc as plsc

V = 1000000
D = 64
B = 16384
NC = 2
NS = 16
NW = NC * NS
BPW = B // NW
L = 16


def _glove_body(ctx_hbm, tgt_hbm, wt_hbm, bt_hbm, wc_hbm, bc_hbm, out_hbm,
                tidx, cidx, wt, wc, bt, bc, acc, sem):
    wid = lax.axis_index("s") * NC + lax.axis_index("c")
    base = wid * BPW

    pltpu.sync_copy(tgt_hbm.at[pl.ds(base, BPW)], tidx)
    pltpu.sync_copy(ctx_hbm.at[pl.ds(base, BPW)], cidx)

    cp_wt = pltpu.async_copy(wt_hbm.at[tidx], wt, sem)
    cp_wc = pltpu.async_copy(wc_hbm.at[cidx], wc, sem)
    cp_bt = pltpu.async_copy(bt_hbm.at[tidx], bt, sem)
    cp_bc = pltpu.async_copy(bc_hbm.at[cidx], bc, sem)
    cp_wt.wait()
    cp_wc.wait()
    cp_bt.wait()
    cp_bc.wait()

    lane = lax.iota(jnp.int32, L)
    last_lane = lane == (L - 1)

    def row(r, carry):
        p = wt[r, pl.ds(0, L)] * wc[r, pl.ds(0, L)]
        p = p + wt[r, pl.ds(L, L)] * wc[r, pl.ds(L, L)]
        p = p + wt[r, pl.ds(2 * L, L)] * wc[r, pl.ds(2 * L, L)]
        p = p + wt[r, pl.ds(3 * L, L)] * wc[r, pl.ds(3 * L, L)]
        s = plsc.cumsum(p)
        plsc.store_scatter(acc, [jnp.full((L,), r, jnp.int32)], s,
                           mask=last_lane)
        return carry

    lax.fori_loop(0, BPW, row, 0, unroll=8)

    for i in range(BPW // L):
        sl = pl.ds(i * L, L)
        acc[sl] = acc[sl] + bt[sl] + bc[sl]

    pltpu.sync_copy(acc, out_hbm.at[pl.ds(base, BPW)])


@jax.jit
def _glove_sc(context_input, target_input, W_target, b_target_flat,
              W_context, b_context_flat):
    mesh = plsc.VectorSubcoreMesh(core_axis_name="c", subcore_axis_name="s")
    return pl.kernel(
        _glove_body,
        mesh=mesh,
        compiler_params=pltpu.CompilerParams(
            needs_layout_passes=False, use_tc_tiling_on_sc=False),
        out_type=jax.ShapeDtypeStruct((B,), jnp.float32),
        scratch_types=[
            pltpu.VMEM((BPW,), jnp.int32),
            pltpu.VMEM((BPW,), jnp.int32),
            pltpu.VMEM((BPW, D), jnp.float32),
            pltpu.VMEM((BPW, D), jnp.float32),
            pltpu.VMEM((BPW,), jnp.float32),
            pltpu.VMEM((BPW,), jnp.float32),
            pltpu.VMEM((BPW,), jnp.float32),
            pltpu.SemaphoreType.DMA,
        ],
    )(context_input, target_input, W_target, b_target_flat,
      W_context, b_context_flat)


def kernel(context_input, target_input, W_target, b_target, W_context,
           b_context):
    return _glove_sc(
        context_input.astype(jnp.int32),
        target_input.astype(jnp.int32),
        W_target,
        jnp.reshape(b_target, (V,)),
        W_context,
        jnp.reshape(b_context, (V,)),
    )
